# baseline (speedup 1.0000x reference)
import functools

import jax
import jax.numpy as jnp
from jax import lax
from jax.experimental import pallas as pl
from jax.experimental.pallas import tpu as pltpu
from jax.experimental.pallas import tpu_sc as plsc

NC = 2
NS = 16
NW = NC * NS
CHUNK = 128
NSUB = 4
SUP = CHUNK * NSUB


@functools.lru_cache(maxsize=None)
def _build(V, D, B):
    b_per_w = B // NW
    nch = b_per_w // CHUNK
    nsup = b_per_w // SUP
    mesh = plsc.VectorSubcoreMesh(core_axis_name="c", subcore_axis_name="s")

    @functools.partial(
        pl.kernel,
        mesh=mesh,
        compiler_params=pltpu.CompilerParams(use_tc_tiling_on_sc=False),
        out_type=jax.ShapeDtypeStruct((B, D), jnp.float32),
        scratch_types=[
            pltpu.VMEM((nch, CHUNK), jnp.int32),
            pltpu.VMEM((2, SUP, D), jnp.float32),
            pltpu.SemaphoreType.DMA,
            pltpu.SemaphoreType.DMA,
            pltpu.SemaphoreType.DMA,
            pltpu.SemaphoreType.DMA,
        ],
    )
    def emb_kernel(table_hbm, idx_hbm, out_hbm, idx_v, rows_v, g0, g1, o0, o1):
        wid = lax.axis_index("s") * NC + lax.axis_index("c")
        base = wid * b_per_w
        pltpu.sync_copy(idx_hbm.at[wid], idx_v)

        def fire(s, b, gsem):
            waits = []
            for j in range(NSUB):
                cp = pltpu.async_copy(
                    table_hbm.at[idx_v.at[s * NSUB + j]],
                    rows_v.at[b, pl.ds(j * CHUNK, CHUNK), :],
                    gsem,
                )
                waits.append(cp)
            return waits

        def out_start(s, b, osem):
            pltpu.async_copy(
                rows_v.at[b], out_hbm.at[pl.ds(base + s * SUP, SUP), :], osem
            )

        def out_wait(b, osem):
            pltpu.make_async_copy(
                table_hbm.at[pl.ds(0, SUP), :], rows_v.at[b], osem
            ).wait()

        w0 = fire(0, 0, g0)
        w1 = fire(1, 1, g1)
        for w in w0:
            w.wait()
        out_start(0, 0, o0)
        for w in w1:
            w.wait()
        out_start(1, 1, o1)

        def loop_body(k, carry):
            s0 = 2 * k + 2
            s1 = 2 * k + 3
            out_wait(0, o0)
            wa = fire(s0, 0, g0)
            out_wait(1, o1)
            wb = fire(s1, 1, g1)
            for w in wa:
                w.wait()
            out_start(s0, 0, o0)
            for w in wb:
                w.wait()
            out_start(s1, 1, o1)
            return carry

        lax.fori_loop(0, nsup // 2 - 1, loop_body, 0)
        out_wait(0, o0)
        out_wait(1, o1)

    return emb_kernel


def kernel(input_ids, table):
    batch, seq = input_ids.shape
    V, D = table.shape
    B = batch * seq
    idx = input_ids.astype(jnp.int32).reshape(NW, B // NW // CHUNK, CHUNK)
    out = _build(V, D, B)(table, idx)
    return out.reshape(batch, seq, D)


# speedup vs baseline: 1.3313x; 1.3313x over previous
import functools

import jax
import jax.numpy as jnp
from jax import lax
from jax.experimental import pallas as pl
from jax.experimental.pallas import tpu as pltpu
from jax.experimental.pallas import tpu_sc as plsc

NC = 2
NS = 16
NW = NC * NS
CHUNK = 128
NSUB = 4
SUP = CHUNK * NSUB
PAD = 128


@functools.lru_cache(maxsize=None)
def _build(V, D, B):
    b_per_w = B // NW
    nch = b_per_w // CHUNK
    nsup = b_per_w // SUP
    mesh = plsc.VectorSubcoreMesh(core_axis_name="c", subcore_axis_name="s")

    @functools.partial(
        pl.kernel,
        mesh=mesh,
        compiler_params=pltpu.CompilerParams(use_tc_tiling_on_sc=False),
        out_type=jax.ShapeDtypeStruct((B, PAD), jnp.float32),
        scratch_types=[
            pltpu.VMEM((nch, CHUNK), jnp.int32),
            pltpu.VMEM((2, SUP, D), jnp.float32),
            pltpu.SemaphoreType.DMA,
            pltpu.SemaphoreType.DMA,
            pltpu.SemaphoreType.DMA,
            pltpu.SemaphoreType.DMA,
        ],
    )
    def emb_kernel(table_hbm, idx_hbm, out_hbm, idx_v, rows_v, g0, g1, o0, o1):
        wid = lax.axis_index("s") * NC + lax.axis_index("c")
        base = wid * b_per_w
        pltpu.sync_copy(idx_hbm.at[wid], idx_v)

        def fire(s, b, gsem):
            waits = []
            for j in range(NSUB):
                cp = pltpu.async_copy(
                    table_hbm.at[idx_v.at[s * NSUB + j]],
                    rows_v.at[b, pl.ds(j * CHUNK, CHUNK), :],
                    gsem,
                )
                waits.append(cp)
            return waits

        def out_start(s, b, osem):
            pltpu.async_copy(
                rows_v.at[b],
                out_hbm.at[pl.ds(base + s * SUP, SUP), pl.ds(0, D)],
                osem,
            )

        def out_wait(b, osem):
            pltpu.make_async_copy(
                table_hbm.at[pl.ds(0, SUP), :], rows_v.at[b], osem
            ).wait()

        w0 = fire(0, 0, g0)
        w1 = fire(1, 1, g1)
        for w in w0:
            w.wait()
        out_start(0, 0, o0)
        for w in w1:
            w.wait()
        out_start(1, 1, o1)

        def loop_body(k, carry):
            s0 = 2 * k + 2
            s1 = 2 * k + 3
            out_wait(0, o0)
            wa = fire(s0, 0, g0)
            out_wait(1, o1)
            wb = fire(s1, 1, g1)
            for w in wa:
                w.wait()
            out_start(s0, 0, o0)
            for w in wb:
                w.wait()
            out_start(s1, 1, o1)
            return carry

        lax.fori_loop(0, nsup // 2 - 1, loop_body, 0)
        out_wait(0, o0)
        out_wait(1, o1)

    return emb_kernel


def kernel(input_ids, table):
    batch, seq = input_ids.shape
    V, D = table.shape
    B = batch * seq
    idx = input_ids.astype(jnp.int32).reshape(NW, B // NW // CHUNK, CHUNK)
    out = _build(V, D, B)(table, idx)
    return out[:, :D].reshape(batch, seq, D)


# speedup vs baseline: 1.3343x; 1.0022x over previous
import functools

import jax
import jax.numpy as jnp
from jax import lax
from jax.experimental import pallas as pl
from jax.experimental.pallas import tpu as pltpu
from jax.experimental.pallas import tpu_sc as plsc

NC = 2
NS = 16
NW = NC * NS
CHUNK = 128
NSUB = 4
SUP = CHUNK * NSUB
PAD = 128


@functools.lru_cache(maxsize=None)
def _build(V, D, B):
    b_per_w = B // NW
    nch = b_per_w // CHUNK
    nsup = b_per_w // SUP
    mesh = plsc.VectorSubcoreMesh(core_axis_name="c", subcore_axis_name="s")

    @functools.partial(
        pl.kernel,
        mesh=mesh,
        compiler_params=pltpu.CompilerParams(use_tc_tiling_on_sc=False),
        out_type=jax.ShapeDtypeStruct((B, PAD), jnp.float32),
        scratch_types=[
            pltpu.VMEM((nch, CHUNK), jnp.int32),
            pltpu.VMEM((3, SUP, D), jnp.float32),
            pltpu.SemaphoreType.DMA,
            pltpu.SemaphoreType.DMA,
            pltpu.SemaphoreType.DMA,
            pltpu.SemaphoreType.DMA,
            pltpu.SemaphoreType.DMA,
            pltpu.SemaphoreType.DMA,
        ],
    )
    def emb_kernel(
        table_hbm, idx_hbm, out_hbm, idx_v, rows_v, g0, g1, g2, o0, o1, o2
    ):
        wid = lax.axis_index("s") * NC + lax.axis_index("c")
        base = wid * b_per_w
        pltpu.sync_copy(idx_hbm.at[wid], idx_v)

        def fire(s, b, gsem):
            waits = []
            for j in range(NSUB):
                cp = pltpu.async_copy(
                    table_hbm.at[idx_v.at[s * NSUB + j]],
                    rows_v.at[b, pl.ds(j * CHUNK, CHUNK), :],
                    gsem,
                )
                waits.append(cp)
            return waits

        def out_start(s, b, osem):
            pltpu.async_copy(
                rows_v.at[b],
                out_hbm.at[pl.ds(base + s * SUP, SUP), pl.ds(0, D)],
                osem,
            )

        def out_wait(b, osem):
            pltpu.make_async_copy(
                table_hbm.at[pl.ds(0, SUP), :], rows_v.at[b], osem
            ).wait()

        w0 = fire(0, 0, g0)
        w1 = fire(1, 1, g1)
        for w in w0:
            w.wait()
        out_start(0, 0, o0)
        for w in w1:
            w.wait()
        out_start(1, 1, o1)

        wa = fire(2, 2, g2)
        out_wait(0, o0)
        wb = fire(3, 0, g0)
        out_wait(1, o1)
        wc = fire(4, 1, g1)
        for w in wa:
            w.wait()
        out_start(2, 2, o2)
        for w in wb:
            w.wait()
        out_start(3, 0, o0)
        for w in wc:
            w.wait()
        out_start(4, 1, o1)

        def loop_body(k, carry):
            s0 = 3 * k + 2
            s1 = 3 * k + 3
            s2 = 3 * k + 4
            out_wait(2, o2)
            wa = fire(s0, 2, g2)
            out_wait(0, o0)
            wb = fire(s1, 0, g0)
            out_wait(1, o1)
            wc = fire(s2, 1, g1)
            for w in wa:
                w.wait()
            out_start(s0, 2, o2)
            for w in wb:
                w.wait()
            out_start(s1, 0, o0)
            for w in wc:
                w.wait()
            out_start(s2, 1, o1)
            return carry

        lax.fori_loop(1, (nsup - 2) // 3, loop_body, 0)
        out_wait(2, o2)
        out_wait(0, o0)
        out_wait(1, o1)

    return emb_kernel


def kernel(input_ids, table):
    batch, seq = input_ids.shape
    V, D = table.shape
    B = batch * seq
    idx = input_ids.astype(jnp.int32).reshape(NW, B // NW // CHUNK, CHUNK)
    out = _build(V, D, B)(table, idx)
    return out[:, :D].reshape(batch, seq, D)
